# baseline (speedup 1.0000x reference)
import jax
import jax.numpy as jnp
from jax import lax
from jax.experimental import pallas as pl
from jax.experimental.pallas import tpu as pltpu
from jax.experimental.pallas import tpu_sc as plsc

_NC = 2
_NS = 16
_NW = _NC * _NS


def _sc_gather(ctab, idx_flat):
    v = ctab.shape[0]
    total = idx_flat.shape[0]
    per_tile = total // _NW
    n_chunks = max(1, -(-per_tile // 1536))
    while per_tile % n_chunks:
        n_chunks += 1
    chunk = per_tile // n_chunks

    mesh = plsc.VectorSubcoreMesh(
        core_axis_name="c", subcore_axis_name="s",
        num_cores=_NC, num_subcores=_NS)

    def body(ctab_hbm, idx_hbm, out_hbm, tab_v, idx_v, rows_v, osem):
        wid = lax.axis_index("s") * _NC + lax.axis_index("c")
        base = wid * per_tile
        pltpu.sync_copy(ctab_hbm, tab_v)
        iota16 = lax.iota(jnp.int32, 16)
        rots = [jnp.bitwise_and(iota16 + d, 15) for d in range(16)]
        posd = [iota16 * 16 + rots[d] for d in range(16)]

        def wb_slice(c):
            return out_hbm.at[pl.ds((base + c * chunk) * 16, chunk * 16)]

        for c in range(n_chunks):
            cur = c % 2
            if c >= 2:
                pltpu.make_async_copy(rows_v.at[cur], wb_slice(c - 2),
                                      osem).wait()
            pltpu.sync_copy(idx_hbm.at[pl.ds(base + c * chunk, chunk)],
                            idx_v)
            rv = rows_v.at[cur]

            def bg(g, carry):
                addr = idx_v[pl.ds(g * 16, 16)] * 16
                ob = g * 256
                for d in range(16):
                    vals = plsc.load_gather(tab_v, [addr + rots[d]])
                    plsc.store_scatter(rv, [posd[d] + ob], vals)
                return carry

            lax.fori_loop(0, chunk // 16, bg, 0, unroll=8)
            pltpu.async_copy(rv, wb_slice(c), osem)
        pltpu.make_async_copy(rows_v.at[(n_chunks - 1) % 2],
                              wb_slice(n_chunks - 1), osem).wait()
        if n_chunks >= 2:
            pltpu.make_async_copy(rows_v.at[n_chunks % 2],
                                  wb_slice(n_chunks - 2), osem).wait()

    f = pl.kernel(
        body,
        out_type=jax.ShapeDtypeStruct((total * 16,), jnp.float32),
        mesh=mesh,
        scratch_types=[
            pltpu.VMEM((v * 16,), jnp.float32),
            pltpu.VMEM((chunk,), jnp.int32),
            pltpu.VMEM((2, chunk * 16), jnp.float32),
            pltpu.SemaphoreType.DMA,
        ],
        compiler_params=pltpu.CompilerParams(use_tc_tiling_on_sc=False,
                                             needs_layout_passes=False),
    )
    return f(ctab.reshape(-1), idx_flat)


def _tc_body(g_ref, hp_ref, lvl_ref, att_ref, defn_ref, spe_ref, spA_ref,
             spD_ref, pp_ref, exp_ref, ehp_ref, elvl_ref, oh_ref,
             plvl_ref, phl_ref, inb_ref, badge_ref, hms_ref, map_ref,
             pW1e_ref, G2_ref, pb1_ref, pW2_ref, pb2_ref,
             eb1_ref, eW2_ref, eb2_ref,
             paW_ref, pab_ref, gW_ref, gb_ref,
             fWp_ref, fWe_ref, fWpa_ref, fWm_ref, fWg_ref, fb_ref,
             out_ref):
    n = out_ref.shape[0]
    pW1e = pW1e_ref[...]
    pb1 = pb1_ref[...]
    pW2 = pW2_ref[...]
    eb1 = eb1_ref[...]
    eW2 = eW2_ref[...]

    r = jnp.concatenate(
        [hp_ref[...], lvl_ref[...], att_ref[...], defn_ref[...],
         spe_ref[...], spA_ref[...], spD_ref[...], pp_ref[...],
         exp_ref[...], ehp_ref[...], elvl_ref[...], oh_ref[...]],
        axis=1)

    pacc = jnp.zeros((n, 128), jnp.float32)
    eacc = jnp.zeros((n, 128), jnp.float32)
    for p in range(6):
        emb = g_ref[p]
        sb = jnp.dot(r, G2_ref[p], preferred_element_type=jnp.float32)
        h1 = jax.nn.relu(
            jnp.dot(emb, pW1e, preferred_element_type=jnp.float32)
            + sb[:, 0:128] + pb1)
        pacc = pacc + jnp.dot(h1, pW2, preferred_element_type=jnp.float32)

        g1 = jax.nn.relu(sb[:, 128:256] + eb1)
        eacc = eacc + jnp.dot(g1, eW2, preferred_element_type=jnp.float32)

    player = pacc * (1.0 / 6.0) + pb2_ref[...]
    enemy = eacc * (1.0 / 6.0) + eb2_ref[...]

    php = jnp.mean(phl_ref[...], axis=1, keepdims=True)
    plv = jnp.mean(plvl_ref[...].astype(jnp.float32) / 100.0, axis=1,
                   keepdims=True)
    party = php * paW_ref[0:1, :] + plv * paW_ref[1:2, :] + pab_ref[...]

    g_in = jnp.concatenate([inb_ref[...], badge_ref[...], hms_ref[...]],
                           axis=1)
    gv = jax.nn.relu(jnp.dot(g_in, gW_ref[...],
                             preferred_element_type=jnp.float32)
                     + gb_ref[...])

    out = (jnp.dot(player, fWp_ref[...], preferred_element_type=jnp.float32)
           + jnp.dot(enemy, fWe_ref[...], preferred_element_type=jnp.float32)
           + jnp.dot(party, fWpa_ref[...], preferred_element_type=jnp.float32)
           + jnp.dot(map_ref[...], fWm_ref[...],
                     preferred_element_type=jnp.float32)
           + jnp.dot(gv, fWg_ref[...], preferred_element_type=jnp.float32)
           + fb_ref[...])
    out_ref[...] = jax.nn.relu(out)


def kernel(p_species, p_moves, p_ability, p_status, p_item, e_status,
           party_level, p_hp, p_lvl, p_att, p_defn, p_spe, p_spA, p_spD,
           p_pp, p_exp, e_hp, e_lvl, party_hp, inbattle, badge, hms,
           map_feat, species_emb, move_emb, ability_emb, status_emb,
           item_emb, e_status_emb, pW1, pb1, pW2, pb2, eW1, eb1, eW2, eb2,
           partyW, partyb, gW, gb, fW, fb):
    b = p_species.shape[0]
    f32 = jnp.float32

    ctab = jnp.concatenate([species_emb, move_emb, ability_emb, status_emb,
                            item_emb], axis=0)
    idx_px = jnp.concatenate([
        p_species[..., None].astype(jnp.int32),
        p_moves.astype(jnp.int32) + 412,
        p_ability[..., None].astype(jnp.int32) + 767,
        p_status[..., None].astype(jnp.int32) + 845,
        p_item[..., None].astype(jnp.int32) + 851,
    ], axis=-1)

    oh36 = (e_status[..., None] == jnp.arange(6)).astype(f32).reshape(b, 36)

    eye6 = jnp.eye(6, dtype=f32)

    def blk(mask, w):
        return mask[:, :, None] * w[None, None, :]

    z6 = jnp.zeros((6, 6, 128), f32)
    Gp = jnp.concatenate([
        blk(eye6, pW1[128]), blk(eye6, pW1[129] / 100.0),
        blk(eye6, pW1[130]), blk(eye6, pW1[131]), blk(eye6, pW1[132]),
        blk(eye6, pW1[133]), blk(eye6, pW1[134]),
        blk(jnp.repeat(eye6, 4, axis=1), pW1[135] / 4.0),
        blk(eye6, pW1[136]), z6, z6,
        jnp.zeros((6, 36, 128), f32)], axis=1)
    Test = jnp.dot(e_status_emb, eW1[0:16],
                   preferred_element_type=f32)
    Tblk = (eye6[:, :, None, None]
            * Test[None, None, :, :]).reshape(6, 36, 128)
    Ge = jnp.concatenate([
        jnp.zeros((6, 72, 128), f32),
        blk(eye6, eW1[16]), blk(eye6, eW1[17] / 100.0),
        Tblk], axis=1)
    G2 = jnp.concatenate([Gp, Ge], axis=2)

    weights = [pW1[0:128], G2, pb1.reshape(1, 128), pW2,
               pb2.reshape(1, 128),
               eb1.reshape(1, 128), eW2, eb2.reshape(1, 128),
               partyW, partyb.reshape(1, 128), gW, gb.reshape(1, 32),
               fW[0:128], fW[128:256], fW[256:384], fW[384:439],
               fW[439:471], fb.reshape(1, 256)]

    batch_in = [(p_hp, 6), (p_lvl, 6), (p_att, 6), (p_defn, 6), (p_spe, 6),
                (p_spA, 6), (p_spD, 6), (p_pp.reshape(b, 24), 24),
                (p_exp, 6), (e_hp, 6), (e_lvl, 6), (oh36, 36),
                (party_level.astype(jnp.int32), 6), (party_hp, 6),
                (inbattle, 1), (badge, 8), (hms, 8), (map_feat, 55)]

    def tc_half(gpx_h, batch_h, bh):
        n = min(512, bh)
        grid = (bh // n,)

        def bspec(k):
            return pl.BlockSpec((n, k), lambda i: (i, 0))

        def wspec(shape):
            nd = len(shape)
            return pl.BlockSpec(shape, lambda i: (0,) * nd)

        in_specs = ([pl.BlockSpec((6, n, 128), lambda i: (0, i, 0))]
                    + [bspec(k) for _, k in batch_in]
                    + [wspec(w.shape) for w in weights])
        return pl.pallas_call(
            _tc_body,
            grid=grid,
            in_specs=in_specs,
            out_specs=pl.BlockSpec((n, 256), lambda i: (i, 0)),
            out_shape=jax.ShapeDtypeStruct((bh, 256), f32),
            compiler_params=pltpu.CompilerParams(
                dimension_semantics=("parallel",)),
        )(gpx_h, *batch_h, *weights)

    nsplit = 2 if b % 1024 == 0 else 1
    bh = b // nsplit
    outs = []
    gpxs = []
    for s in range(nsplit):
        idx_h = idx_px[s * bh:(s + 1) * bh]
        idx_h = idx_h.transpose(1, 0, 2).reshape(-1)
        gpxs.append(_sc_gather(ctab, idx_h).reshape(6, bh, 128))
    for s in range(nsplit):
        batch_h = [a[s * bh:(s + 1) * bh] for a, _ in batch_in]
        outs.append(tc_half(gpxs[s], batch_h, bh))
    if nsplit == 1:
        return outs[0]
    return jnp.concatenate(outs, axis=0)


# speedup vs baseline: 1.1212x; 1.1212x over previous
import jax
import jax.numpy as jnp
from jax import lax
from jax.experimental import pallas as pl
from jax.experimental.pallas import tpu as pltpu
from jax.experimental.pallas import tpu_sc as plsc

_NC = 2
_NS = 16
_NW = _NC * _NS


def _sc_gather(ctab, idx_flat):
    v = ctab.shape[0]
    total = idx_flat.shape[0]
    per_tile = total // _NW
    n_chunks = max(1, -(-per_tile // 1536))
    while per_tile % n_chunks:
        n_chunks += 1
    chunk = per_tile // n_chunks

    mesh = plsc.VectorSubcoreMesh(
        core_axis_name="c", subcore_axis_name="s",
        num_cores=_NC, num_subcores=_NS)

    def body(ctab_hbm, idx_hbm, out_hbm, tab_v, idx_v, rows_v, osem):
        wid = lax.axis_index("s") * _NC + lax.axis_index("c")
        base = wid * per_tile
        pltpu.sync_copy(ctab_hbm, tab_v)
        iota16 = lax.iota(jnp.int32, 16)
        rots = [jnp.bitwise_and(iota16 + d, 15) for d in range(16)]
        posd = [iota16 * 16 + rots[d] for d in range(16)]

        def wb_slice(c):
            return out_hbm.at[pl.ds((base + c * chunk) * 16, chunk * 16)]

        for c in range(n_chunks):
            cur = c % 2
            if c >= 2:
                pltpu.make_async_copy(rows_v.at[cur], wb_slice(c - 2),
                                      osem).wait()
            pltpu.sync_copy(idx_hbm.at[pl.ds(base + c * chunk, chunk)],
                            idx_v)
            rv = rows_v.at[cur]

            def bg(g, carry):
                addr = idx_v[pl.ds(g * 16, 16)] * 16
                ob = g * 256
                for d in range(16):
                    vals = plsc.load_gather(tab_v, [addr + rots[d]])
                    plsc.store_scatter(rv, [posd[d] + ob], vals)
                return carry

            lax.fori_loop(0, chunk // 16, bg, 0, unroll=8)
            pltpu.async_copy(rv, wb_slice(c), osem)
        pltpu.make_async_copy(rows_v.at[(n_chunks - 1) % 2],
                              wb_slice(n_chunks - 1), osem).wait()
        if n_chunks >= 2:
            pltpu.make_async_copy(rows_v.at[n_chunks % 2],
                                  wb_slice(n_chunks - 2), osem).wait()

    f = pl.kernel(
        body,
        out_type=jax.ShapeDtypeStruct((total * 16,), jnp.float32),
        mesh=mesh,
        scratch_types=[
            pltpu.VMEM((v * 16,), jnp.float32),
            pltpu.VMEM((chunk,), jnp.int32),
            pltpu.VMEM((2, chunk * 16), jnp.float32),
            pltpu.SemaphoreType.DMA,
        ],
        compiler_params=pltpu.CompilerParams(use_tc_tiling_on_sc=False,
                                             needs_layout_passes=False),
    )
    return f(ctab.reshape(-1), idx_flat)


def _tc_body(g_ref, hp_ref, lvl_ref, att_ref, defn_ref, spe_ref, spA_ref,
             spD_ref, pp_ref, exp_ref, ehp_ref, elvl_ref, oh_ref,
             plvl_ref, phl_ref, inb_ref, badge_ref, hms_ref, map_ref,
             pW1e_ref, G2_ref, pb1_ref, pW2_ref, pb2_ref,
             eb1_ref, eW2_ref, eb2_ref,
             paW_ref, pab_ref, gW_ref, gb_ref,
             fWp_ref, fWe_ref, fWpa_ref, fWm_ref, fWg_ref, fb_ref,
             out_ref):
    n = out_ref.shape[0]
    pW1e = pW1e_ref[...]
    pb1 = pb1_ref[...]
    pW2 = pW2_ref[...]
    eb1 = eb1_ref[...]
    eW2 = eW2_ref[...]

    r = jnp.concatenate(
        [hp_ref[...], lvl_ref[...], att_ref[...], defn_ref[...],
         spe_ref[...], spA_ref[...], spD_ref[...], pp_ref[...],
         exp_ref[...], ehp_ref[...], elvl_ref[...], oh_ref[...]],
        axis=1)

    pacc = jnp.zeros((n, 128), jnp.float32)
    eacc = jnp.zeros((n, 128), jnp.float32)
    for p in range(6):
        emb = g_ref[p]
        sb = jnp.dot(r, G2_ref[p], preferred_element_type=jnp.float32)
        h1 = jax.nn.relu(
            jnp.dot(emb, pW1e, preferred_element_type=jnp.float32)
            + sb[:, 0:128] + pb1)
        pacc = pacc + jnp.dot(h1, pW2, preferred_element_type=jnp.float32)

        g1 = jax.nn.relu(sb[:, 128:256] + eb1)
        eacc = eacc + jnp.dot(g1, eW2, preferred_element_type=jnp.float32)

    player = pacc * (1.0 / 6.0) + pb2_ref[...]
    enemy = eacc * (1.0 / 6.0) + eb2_ref[...]

    php = jnp.mean(phl_ref[...], axis=1, keepdims=True)
    plv = jnp.mean(plvl_ref[...].astype(jnp.float32) / 100.0, axis=1,
                   keepdims=True)
    party = php * paW_ref[0:1, :] + plv * paW_ref[1:2, :] + pab_ref[...]

    g_in = jnp.concatenate([inb_ref[...], badge_ref[...], hms_ref[...]],
                           axis=1)
    gv = jax.nn.relu(jnp.dot(g_in, gW_ref[...],
                             preferred_element_type=jnp.float32)
                     + gb_ref[...])

    out = (jnp.dot(player, fWp_ref[...], preferred_element_type=jnp.float32)
           + jnp.dot(enemy, fWe_ref[...], preferred_element_type=jnp.float32)
           + jnp.dot(party, fWpa_ref[...], preferred_element_type=jnp.float32)
           + jnp.dot(map_ref[...], fWm_ref[...],
                     preferred_element_type=jnp.float32)
           + jnp.dot(gv, fWg_ref[...], preferred_element_type=jnp.float32)
           + fb_ref[...])
    out_ref[...] = jax.nn.relu(out)


def kernel(p_species, p_moves, p_ability, p_status, p_item, e_status,
           party_level, p_hp, p_lvl, p_att, p_defn, p_spe, p_spA, p_spD,
           p_pp, p_exp, e_hp, e_lvl, party_hp, inbattle, badge, hms,
           map_feat, species_emb, move_emb, ability_emb, status_emb,
           item_emb, e_status_emb, pW1, pb1, pW2, pb2, eW1, eb1, eW2, eb2,
           partyW, partyb, gW, gb, fW, fb):
    b = p_species.shape[0]
    f32 = jnp.float32

    ctab = jnp.concatenate([species_emb, move_emb, ability_emb, status_emb,
                            item_emb], axis=0)
    idx_px = jnp.concatenate([
        p_species[..., None].astype(jnp.int32),
        p_moves.astype(jnp.int32) + 412,
        p_ability[..., None].astype(jnp.int32) + 767,
        p_status[..., None].astype(jnp.int32) + 845,
        p_item[..., None].astype(jnp.int32) + 851,
    ], axis=-1)

    oh36 = (e_status[..., None] == jnp.arange(6)).astype(f32).reshape(b, 36)

    eye6 = jnp.eye(6, dtype=f32)

    def blk(mask, w):
        return mask[:, :, None] * w[None, None, :]

    z6 = jnp.zeros((6, 6, 128), f32)
    Gp = jnp.concatenate([
        blk(eye6, pW1[128]), blk(eye6, pW1[129] / 100.0),
        blk(eye6, pW1[130]), blk(eye6, pW1[131]), blk(eye6, pW1[132]),
        blk(eye6, pW1[133]), blk(eye6, pW1[134]),
        blk(jnp.repeat(eye6, 4, axis=1), pW1[135] / 4.0),
        blk(eye6, pW1[136]), z6, z6,
        jnp.zeros((6, 36, 128), f32)], axis=1)
    Test = jnp.dot(e_status_emb, eW1[0:16],
                   preferred_element_type=f32)
    Tblk = (eye6[:, :, None, None]
            * Test[None, None, :, :]).reshape(6, 36, 128)
    Ge = jnp.concatenate([
        jnp.zeros((6, 72, 128), f32),
        blk(eye6, eW1[16]), blk(eye6, eW1[17] / 100.0),
        Tblk], axis=1)
    G2 = jnp.concatenate([Gp, Ge], axis=2)

    weights = [pW1[0:128], G2, pb1.reshape(1, 128), pW2,
               pb2.reshape(1, 128),
               eb1.reshape(1, 128), eW2, eb2.reshape(1, 128),
               partyW, partyb.reshape(1, 128), gW, gb.reshape(1, 32),
               fW[0:128], fW[128:256], fW[256:384], fW[384:439],
               fW[439:471], fb.reshape(1, 256)]

    batch_in = [(p_hp, 6), (p_lvl, 6), (p_att, 6), (p_defn, 6), (p_spe, 6),
                (p_spA, 6), (p_spD, 6), (p_pp.reshape(b, 24), 24),
                (p_exp, 6), (e_hp, 6), (e_lvl, 6), (oh36, 36),
                (party_level.astype(jnp.int32), 6), (party_hp, 6),
                (inbattle, 1), (badge, 8), (hms, 8), (map_feat, 55)]

    def tc_half(gpx_h, batch_h, bh):
        n = min(512, bh)
        grid = (bh // n,)

        def bspec(k):
            return pl.BlockSpec((n, k), lambda i: (i, 0))

        def wspec(shape):
            nd = len(shape)
            return pl.BlockSpec(shape, lambda i: (0,) * nd)

        in_specs = ([pl.BlockSpec((6, n, 128), lambda i: (0, i, 0))]
                    + [bspec(k) for _, k in batch_in]
                    + [wspec(w.shape) for w in weights])
        return pl.pallas_call(
            _tc_body,
            grid=grid,
            in_specs=in_specs,
            out_specs=pl.BlockSpec((n, 256), lambda i: (i, 0)),
            out_shape=jax.ShapeDtypeStruct((bh, 256), f32),
            compiler_params=pltpu.CompilerParams(
                dimension_semantics=("parallel",)),
        )(gpx_h, *batch_h, *weights)

    nsplit = 1
    bh = b // nsplit
    outs = []
    gpxs = []
    for s in range(nsplit):
        idx_h = idx_px[s * bh:(s + 1) * bh]
        idx_h = idx_h.transpose(1, 0, 2).reshape(-1)
        gpxs.append(_sc_gather(ctab, idx_h).reshape(6, bh, 128))
    for s in range(nsplit):
        batch_h = [a[s * bh:(s + 1) * bh] for a, _ in batch_in]
        outs.append(tc_half(gpxs[s], batch_h, bh))
    if nsplit == 1:
        return outs[0]
    return jnp.concatenate(outs, axis=0)


# speedup vs baseline: 1.1360x; 1.0131x over previous
import jax
import jax.numpy as jnp
from jax import lax
from jax.experimental import pallas as pl
from jax.experimental.pallas import tpu as pltpu
from jax.experimental.pallas import tpu_sc as plsc

_NC = 2
_NS = 16
_NW = _NC * _NS


def _sc_gather(ctab, idx_flat):
    v = ctab.shape[0]
    total = idx_flat.shape[0]
    per_tile = total // _NW
    n_chunks = max(1, -(-per_tile // 1536))
    while per_tile % n_chunks:
        n_chunks += 1
    chunk = per_tile // n_chunks

    mesh = plsc.VectorSubcoreMesh(
        core_axis_name="c", subcore_axis_name="s",
        num_cores=_NC, num_subcores=_NS)

    def body(ctab_hbm, idx_hbm, out_hbm, tab_v, idx_v, rows_v, osem):
        wid = lax.axis_index("s") * _NC + lax.axis_index("c")
        base = wid * per_tile
        pltpu.sync_copy(ctab_hbm, tab_v)
        iota16 = lax.iota(jnp.int32, 16)
        rots = [jnp.bitwise_and(iota16 + d, 15) for d in range(16)]
        posd = [iota16 * 16 + rots[d] for d in range(16)]

        def wb_slice(c):
            return out_hbm.at[pl.ds((base + c * chunk) * 16, chunk * 16)]

        for c in range(n_chunks):
            cur = c % 2
            if c >= 2:
                pltpu.make_async_copy(rows_v.at[cur], wb_slice(c - 2),
                                      osem).wait()
            pltpu.sync_copy(idx_hbm.at[pl.ds(base + c * chunk, chunk)],
                            idx_v)
            rv = rows_v.at[cur]

            def bg(g, carry):
                addr = idx_v[pl.ds(g * 16, 16)] * 16
                ob = g * 256
                for d in range(16):
                    vals = plsc.load_gather(tab_v, [addr + rots[d]])
                    plsc.store_scatter(rv, [posd[d] + ob], vals)
                return carry

            lax.fori_loop(0, chunk // 16, bg, 0, unroll=4)
            pltpu.async_copy(rv, wb_slice(c), osem)
        pltpu.make_async_copy(rows_v.at[(n_chunks - 1) % 2],
                              wb_slice(n_chunks - 1), osem).wait()
        if n_chunks >= 2:
            pltpu.make_async_copy(rows_v.at[n_chunks % 2],
                                  wb_slice(n_chunks - 2), osem).wait()

    f = pl.kernel(
        body,
        out_type=jax.ShapeDtypeStruct((total * 16,), jnp.float32),
        mesh=mesh,
        scratch_types=[
            pltpu.VMEM((v * 16,), jnp.float32),
            pltpu.VMEM((chunk,), jnp.int32),
            pltpu.VMEM((2, chunk * 16), jnp.float32),
            pltpu.SemaphoreType.DMA,
        ],
        compiler_params=pltpu.CompilerParams(use_tc_tiling_on_sc=False,
                                             needs_layout_passes=False),
    )
    return f(ctab.reshape(-1), idx_flat)


def _tc_body(g_ref, hp_ref, lvl_ref, att_ref, defn_ref, spe_ref, spA_ref,
             spD_ref, pp_ref, exp_ref, ehp_ref, elvl_ref, oh_ref,
             plvl_ref, phl_ref, inb_ref, badge_ref, hms_ref, map_ref,
             pW1e_ref, G2_ref, pb1_ref, pW2_ref, pb2_ref,
             eb1_ref, eW2_ref, eb2_ref,
             paW_ref, pab_ref, gW_ref, gb_ref,
             fWp_ref, fWe_ref, fWpa_ref, fWm_ref, fWg_ref, fb_ref,
             out_ref):
    n = out_ref.shape[0]
    pW1e = pW1e_ref[...]
    pb1 = pb1_ref[...]
    pW2 = pW2_ref[...]
    eb1 = eb1_ref[...]
    eW2 = eW2_ref[...]

    r = jnp.concatenate(
        [hp_ref[...], lvl_ref[...], att_ref[...], defn_ref[...],
         spe_ref[...], spA_ref[...], spD_ref[...], pp_ref[...],
         exp_ref[...], ehp_ref[...], elvl_ref[...], oh_ref[...]],
        axis=1)

    pacc = jnp.zeros((n, 128), jnp.float32)
    eacc = jnp.zeros((n, 128), jnp.float32)
    for p in range(6):
        emb = g_ref[p]
        sb = jnp.dot(r, G2_ref[p], preferred_element_type=jnp.float32)
        h1 = jax.nn.relu(
            jnp.dot(emb, pW1e, preferred_element_type=jnp.float32)
            + sb[:, 0:128] + pb1)
        pacc = pacc + jnp.dot(h1, pW2, preferred_element_type=jnp.float32)

        g1 = jax.nn.relu(sb[:, 128:256] + eb1)
        eacc = eacc + jnp.dot(g1, eW2, preferred_element_type=jnp.float32)

    player = pacc * (1.0 / 6.0) + pb2_ref[...]
    enemy = eacc * (1.0 / 6.0) + eb2_ref[...]

    php = jnp.mean(phl_ref[...], axis=1, keepdims=True)
    plv = jnp.mean(plvl_ref[...].astype(jnp.float32) / 100.0, axis=1,
                   keepdims=True)
    party = php * paW_ref[0:1, :] + plv * paW_ref[1:2, :] + pab_ref[...]

    g_in = jnp.concatenate([inb_ref[...], badge_ref[...], hms_ref[...]],
                           axis=1)
    gv = jax.nn.relu(jnp.dot(g_in, gW_ref[...],
                             preferred_element_type=jnp.float32)
                     + gb_ref[...])

    out = (jnp.dot(player, fWp_ref[...], preferred_element_type=jnp.float32)
           + jnp.dot(enemy, fWe_ref[...], preferred_element_type=jnp.float32)
           + jnp.dot(party, fWpa_ref[...], preferred_element_type=jnp.float32)
           + jnp.dot(map_ref[...], fWm_ref[...],
                     preferred_element_type=jnp.float32)
           + jnp.dot(gv, fWg_ref[...], preferred_element_type=jnp.float32)
           + fb_ref[...])
    out_ref[...] = jax.nn.relu(out)


def kernel(p_species, p_moves, p_ability, p_status, p_item, e_status,
           party_level, p_hp, p_lvl, p_att, p_defn, p_spe, p_spA, p_spD,
           p_pp, p_exp, e_hp, e_lvl, party_hp, inbattle, badge, hms,
           map_feat, species_emb, move_emb, ability_emb, status_emb,
           item_emb, e_status_emb, pW1, pb1, pW2, pb2, eW1, eb1, eW2, eb2,
           partyW, partyb, gW, gb, fW, fb):
    b = p_species.shape[0]
    f32 = jnp.float32

    ctab = jnp.concatenate([species_emb, move_emb, ability_emb, status_emb,
                            item_emb], axis=0)
    idx_px = jnp.concatenate([
        p_species[..., None].astype(jnp.int32),
        p_moves.astype(jnp.int32) + 412,
        p_ability[..., None].astype(jnp.int32) + 767,
        p_status[..., None].astype(jnp.int32) + 845,
        p_item[..., None].astype(jnp.int32) + 851,
    ], axis=-1)

    oh36 = (e_status[..., None] == jnp.arange(6)).astype(f32).reshape(b, 36)

    eye6 = jnp.eye(6, dtype=f32)

    def blk(mask, w):
        return mask[:, :, None] * w[None, None, :]

    z6 = jnp.zeros((6, 6, 128), f32)
    Gp = jnp.concatenate([
        blk(eye6, pW1[128]), blk(eye6, pW1[129] / 100.0),
        blk(eye6, pW1[130]), blk(eye6, pW1[131]), blk(eye6, pW1[132]),
        blk(eye6, pW1[133]), blk(eye6, pW1[134]),
        blk(jnp.repeat(eye6, 4, axis=1), pW1[135] / 4.0),
        blk(eye6, pW1[136]), z6, z6,
        jnp.zeros((6, 36, 128), f32)], axis=1)
    Test = jnp.dot(e_status_emb, eW1[0:16],
                   preferred_element_type=f32)
    Tblk = (eye6[:, :, None, None]
            * Test[None, None, :, :]).reshape(6, 36, 128)
    Ge = jnp.concatenate([
        jnp.zeros((6, 72, 128), f32),
        blk(eye6, eW1[16]), blk(eye6, eW1[17] / 100.0),
        Tblk], axis=1)
    G2 = jnp.concatenate([Gp, Ge], axis=2)

    weights = [pW1[0:128], G2, pb1.reshape(1, 128), pW2,
               pb2.reshape(1, 128),
               eb1.reshape(1, 128), eW2, eb2.reshape(1, 128),
               partyW, partyb.reshape(1, 128), gW, gb.reshape(1, 32),
               fW[0:128], fW[128:256], fW[256:384], fW[384:439],
               fW[439:471], fb.reshape(1, 256)]

    batch_in = [(p_hp, 6), (p_lvl, 6), (p_att, 6), (p_defn, 6), (p_spe, 6),
                (p_spA, 6), (p_spD, 6), (p_pp.reshape(b, 24), 24),
                (p_exp, 6), (e_hp, 6), (e_lvl, 6), (oh36, 36),
                (party_level.astype(jnp.int32), 6), (party_hp, 6),
                (inbattle, 1), (badge, 8), (hms, 8), (map_feat, 55)]

    def tc_half(gpx_h, batch_h, bh):
        n = min(512, bh)
        grid = (bh // n,)

        def bspec(k):
            return pl.BlockSpec((n, k), lambda i: (i, 0))

        def wspec(shape):
            nd = len(shape)
            return pl.BlockSpec(shape, lambda i: (0,) * nd)

        in_specs = ([pl.BlockSpec((6, n, 128), lambda i: (0, i, 0))]
                    + [bspec(k) for _, k in batch_in]
                    + [wspec(w.shape) for w in weights])
        return pl.pallas_call(
            _tc_body,
            grid=grid,
            in_specs=in_specs,
            out_specs=pl.BlockSpec((n, 256), lambda i: (i, 0)),
            out_shape=jax.ShapeDtypeStruct((bh, 256), f32),
            compiler_params=pltpu.CompilerParams(
                dimension_semantics=("parallel",)),
        )(gpx_h, *batch_h, *weights)

    nsplit = 1
    bh = b // nsplit
    outs = []
    gpxs = []
    for s in range(nsplit):
        idx_h = idx_px[s * bh:(s + 1) * bh]
        idx_h = idx_h.transpose(1, 0, 2).reshape(-1)
        gpxs.append(_sc_gather(ctab, idx_h).reshape(6, bh, 128))
    for s in range(nsplit):
        batch_h = [a[s * bh:(s + 1) * bh] for a, _ in batch_in]
        outs.append(tc_half(gpxs[s], batch_h, bh))
    if nsplit == 1:
        return outs[0]
    return jnp.concatenate(outs, axis=0)


# speedup vs baseline: 1.1372x; 1.0010x over previous
import jax
import jax.numpy as jnp
from jax import lax
from jax.experimental import pallas as pl
from jax.experimental.pallas import tpu as pltpu
from jax.experimental.pallas import tpu_sc as plsc

_NC = 2
_NS = 16
_NW = _NC * _NS


def _sc_gather(ctab, idx_flat):
    v = ctab.shape[0]
    total = idx_flat.shape[0]
    per_tile = total // _NW
    n_chunks = max(1, -(-per_tile // 1536))
    while per_tile % n_chunks:
        n_chunks += 1
    chunk = per_tile // n_chunks

    mesh = plsc.VectorSubcoreMesh(
        core_axis_name="c", subcore_axis_name="s",
        num_cores=_NC, num_subcores=_NS)

    def body(ctab_hbm, idx_hbm, out_hbm, tab_v, idx_v, rows_v, osem0,
             osem1):
        osems = (osem0, osem1)
        wid = lax.axis_index("s") * _NC + lax.axis_index("c")
        base = wid * per_tile
        pltpu.sync_copy(ctab_hbm, tab_v)
        iota16 = lax.iota(jnp.int32, 16)
        rots = [jnp.bitwise_and(iota16 + d, 15) for d in range(16)]
        posd = [iota16 * 16 + rots[d] for d in range(16)]

        def wb_slice(c):
            return out_hbm.at[pl.ds((base + c * chunk) * 16, chunk * 16)]

        for c in range(n_chunks):
            cur = c % 2
            if c >= 2:
                pltpu.make_async_copy(rows_v.at[cur], wb_slice(c - 2),
                                      osems[cur]).wait()
            pltpu.sync_copy(idx_hbm.at[pl.ds(base + c * chunk, chunk)],
                            idx_v)
            rv = rows_v.at[cur]

            def bg(g, carry):
                addr = idx_v[pl.ds(g * 16, 16)] * 16
                ob = g * 256
                for d in range(16):
                    vals = plsc.load_gather(tab_v, [addr + rots[d]])
                    plsc.store_scatter(rv, [posd[d] + ob], vals)
                return carry

            lax.fori_loop(0, chunk // 16, bg, 0, unroll=4)
            pltpu.async_copy(rv, wb_slice(c), osems[cur])
        pltpu.make_async_copy(rows_v.at[(n_chunks - 1) % 2],
                              wb_slice(n_chunks - 1),
                              osems[(n_chunks - 1) % 2]).wait()
        if n_chunks >= 2:
            pltpu.make_async_copy(rows_v.at[n_chunks % 2],
                                  wb_slice(n_chunks - 2),
                                  osems[n_chunks % 2]).wait()

    f = pl.kernel(
        body,
        out_type=jax.ShapeDtypeStruct((total * 16,), jnp.float32),
        mesh=mesh,
        scratch_types=[
            pltpu.VMEM((v * 16,), jnp.float32),
            pltpu.VMEM((chunk,), jnp.int32),
            pltpu.VMEM((2, chunk * 16), jnp.float32),
            pltpu.SemaphoreType.DMA,
            pltpu.SemaphoreType.DMA,
        ],
        compiler_params=pltpu.CompilerParams(use_tc_tiling_on_sc=False,
                                             needs_layout_passes=False),
    )
    return f(ctab.reshape(-1), idx_flat)


def _tc_body(g_ref, hp_ref, lvl_ref, att_ref, defn_ref, spe_ref, spA_ref,
             spD_ref, pp_ref, exp_ref, ehp_ref, elvl_ref, oh_ref,
             plvl_ref, phl_ref, inb_ref, badge_ref, hms_ref, map_ref,
             pW1e_ref, G2_ref, pb1_ref, pW2_ref, pb2_ref,
             eb1_ref, eW2_ref, eb2_ref,
             paW_ref, pab_ref, gW_ref, gb_ref,
             fWp_ref, fWe_ref, fWpa_ref, fWm_ref, fWg_ref, fb_ref,
             out_ref):
    n = out_ref.shape[0]
    pW1e = pW1e_ref[...]
    pb1 = pb1_ref[...]
    pW2 = pW2_ref[...]
    eb1 = eb1_ref[...]
    eW2 = eW2_ref[...]

    r = jnp.concatenate(
        [hp_ref[...], lvl_ref[...], att_ref[...], defn_ref[...],
         spe_ref[...], spA_ref[...], spD_ref[...], pp_ref[...],
         exp_ref[...], ehp_ref[...], elvl_ref[...], oh_ref[...]],
        axis=1)

    pacc = jnp.zeros((n, 128), jnp.float32)
    eacc = jnp.zeros((n, 128), jnp.float32)
    for p in range(6):
        emb = g_ref[p]
        sb = jnp.dot(r, G2_ref[p], preferred_element_type=jnp.float32)
        h1 = jax.nn.relu(
            jnp.dot(emb, pW1e, preferred_element_type=jnp.float32)
            + sb[:, 0:128] + pb1)
        pacc = pacc + jnp.dot(h1, pW2, preferred_element_type=jnp.float32)

        g1 = jax.nn.relu(sb[:, 128:256] + eb1)
        eacc = eacc + jnp.dot(g1, eW2, preferred_element_type=jnp.float32)

    player = pacc * (1.0 / 6.0) + pb2_ref[...]
    enemy = eacc * (1.0 / 6.0) + eb2_ref[...]

    php = jnp.mean(phl_ref[...], axis=1, keepdims=True)
    plv = jnp.mean(plvl_ref[...].astype(jnp.float32) / 100.0, axis=1,
                   keepdims=True)
    party = php * paW_ref[0:1, :] + plv * paW_ref[1:2, :] + pab_ref[...]

    g_in = jnp.concatenate([inb_ref[...], badge_ref[...], hms_ref[...]],
                           axis=1)
    gv = jax.nn.relu(jnp.dot(g_in, gW_ref[...],
                             preferred_element_type=jnp.float32)
                     + gb_ref[...])

    out = (jnp.dot(player, fWp_ref[...], preferred_element_type=jnp.float32)
           + jnp.dot(enemy, fWe_ref[...], preferred_element_type=jnp.float32)
           + jnp.dot(party, fWpa_ref[...], preferred_element_type=jnp.float32)
           + jnp.dot(map_ref[...], fWm_ref[...],
                     preferred_element_type=jnp.float32)
           + jnp.dot(gv, fWg_ref[...], preferred_element_type=jnp.float32)
           + fb_ref[...])
    out_ref[...] = jax.nn.relu(out)


def kernel(p_species, p_moves, p_ability, p_status, p_item, e_status,
           party_level, p_hp, p_lvl, p_att, p_defn, p_spe, p_spA, p_spD,
           p_pp, p_exp, e_hp, e_lvl, party_hp, inbattle, badge, hms,
           map_feat, species_emb, move_emb, ability_emb, status_emb,
           item_emb, e_status_emb, pW1, pb1, pW2, pb2, eW1, eb1, eW2, eb2,
           partyW, partyb, gW, gb, fW, fb):
    b = p_species.shape[0]
    f32 = jnp.float32

    ctab = jnp.concatenate([species_emb, move_emb, ability_emb, status_emb,
                            item_emb], axis=0)
    idx_px = jnp.concatenate([
        p_species[..., None].astype(jnp.int32),
        p_moves.astype(jnp.int32) + 412,
        p_ability[..., None].astype(jnp.int32) + 767,
        p_status[..., None].astype(jnp.int32) + 845,
        p_item[..., None].astype(jnp.int32) + 851,
    ], axis=-1)

    oh36 = (e_status[..., None] == jnp.arange(6)).astype(f32).reshape(b, 36)

    eye6 = jnp.eye(6, dtype=f32)

    def blk(mask, w):
        return mask[:, :, None] * w[None, None, :]

    z6 = jnp.zeros((6, 6, 128), f32)
    Gp = jnp.concatenate([
        blk(eye6, pW1[128]), blk(eye6, pW1[129] / 100.0),
        blk(eye6, pW1[130]), blk(eye6, pW1[131]), blk(eye6, pW1[132]),
        blk(eye6, pW1[133]), blk(eye6, pW1[134]),
        blk(jnp.repeat(eye6, 4, axis=1), pW1[135] / 4.0),
        blk(eye6, pW1[136]), z6, z6,
        jnp.zeros((6, 36, 128), f32)], axis=1)
    Test = jnp.dot(e_status_emb, eW1[0:16],
                   preferred_element_type=f32)
    Tblk = (eye6[:, :, None, None]
            * Test[None, None, :, :]).reshape(6, 36, 128)
    Ge = jnp.concatenate([
        jnp.zeros((6, 72, 128), f32),
        blk(eye6, eW1[16]), blk(eye6, eW1[17] / 100.0),
        Tblk], axis=1)
    G2 = jnp.concatenate([Gp, Ge], axis=2)

    weights = [pW1[0:128], G2, pb1.reshape(1, 128), pW2,
               pb2.reshape(1, 128),
               eb1.reshape(1, 128), eW2, eb2.reshape(1, 128),
               partyW, partyb.reshape(1, 128), gW, gb.reshape(1, 32),
               fW[0:128], fW[128:256], fW[256:384], fW[384:439],
               fW[439:471], fb.reshape(1, 256)]

    batch_in = [(p_hp, 6), (p_lvl, 6), (p_att, 6), (p_defn, 6), (p_spe, 6),
                (p_spA, 6), (p_spD, 6), (p_pp.reshape(b, 24), 24),
                (p_exp, 6), (e_hp, 6), (e_lvl, 6), (oh36, 36),
                (party_level.astype(jnp.int32), 6), (party_hp, 6),
                (inbattle, 1), (badge, 8), (hms, 8), (map_feat, 55)]

    def tc_half(gpx_h, batch_h, bh):
        n = min(512, bh)
        grid = (bh // n,)

        def bspec(k):
            return pl.BlockSpec((n, k), lambda i: (i, 0))

        def wspec(shape):
            nd = len(shape)
            return pl.BlockSpec(shape, lambda i: (0,) * nd)

        in_specs = ([pl.BlockSpec((6, n, 128), lambda i: (0, i, 0))]
                    + [bspec(k) for _, k in batch_in]
                    + [wspec(w.shape) for w in weights])
        return pl.pallas_call(
            _tc_body,
            grid=grid,
            in_specs=in_specs,
            out_specs=pl.BlockSpec((n, 256), lambda i: (i, 0)),
            out_shape=jax.ShapeDtypeStruct((bh, 256), f32),
            compiler_params=pltpu.CompilerParams(
                dimension_semantics=("parallel",)),
        )(gpx_h, *batch_h, *weights)

    nsplit = 1
    bh = b // nsplit
    outs = []
    gpxs = []
    for s in range(nsplit):
        idx_h = idx_px[s * bh:(s + 1) * bh]
        idx_h = idx_h.transpose(1, 0, 2).reshape(-1)
        gpxs.append(_sc_gather(ctab, idx_h).reshape(6, bh, 128))
    for s in range(nsplit):
        batch_h = [a[s * bh:(s + 1) * bh] for a, _ in batch_in]
        outs.append(tc_half(gpxs[s], batch_h, bh))
    if nsplit == 1:
        return outs[0]
    return jnp.concatenate(outs, axis=0)
